# Initial kernel scaffold; baseline (speedup 1.0000x reference)
import jax
import jax.numpy as jnp
from jax.experimental import pallas as pl


def kernel(out, target, neighbor_idxs, neighbor_weights):
    raise NotImplementedError("write your pallas kernel here")


# speedup vs baseline: 4.5009x; 4.5009x over previous
import functools

import jax
import jax.numpy as jnp
from jax.experimental import pallas as pl
from jax.experimental.pallas import tpu as pltpu

_OFFSETS = (-2, -1, 1, 2)


def _roll_lanes(x, s):
    n = x.shape[-1]
    s = s % n
    if s == 0:
        return x
    return jnp.concatenate([x[:, s:], x[:, :s]], axis=1)


def _loss_body(v, nb, ni3_ref, nw3_ref, out_ref, tgt_ref, loss_ref, w3_ref):
    i = pl.program_id(0)

    @pl.when(i == 0)
    def _init():
        ni = ni3_ref[...]
        nw = nw3_ref[...]
        p = jax.lax.broadcasted_iota(jnp.int32, ni.shape, 1)
        vert = p // 3
        for slot, o in enumerate(_OFFSETS):
            t = vert + o
            t = jnp.where(t < 0, t + nb, t)
            t = jnp.where(t >= nb, t - nb, t)
            w3_ref[slot : slot + 1, :] = jnp.sum(
                jnp.where(ni == t, nw, 0.0), axis=0, keepdims=True
            )
        loss_ref[0, 0] = 0.0

    d = out_ref[...] - tgt_ref[...]
    acc = d
    for slot, o in enumerate(_OFFSETS):
        acc = acc + w3_ref[slot : slot + 1, :] * _roll_lanes(d, 3 * o)
    loss_ref[0, 0] += jnp.sum(acc * acc)

    @pl.when(i == pl.num_programs(0) - 1)
    def _final():
        loss_ref[0, 0] = loss_ref[0, 0] / v


def kernel(out, target, neighbor_idxs, neighbor_weights):
    b, nb, c = out.shape
    nmax = neighbor_idxs.shape[1]
    n_lanes = nb * c
    out2 = out.reshape(b, n_lanes)
    tgt2 = target.reshape(b, n_lanes)
    ni3 = jnp.repeat(neighbor_idxs.T, c, axis=1)
    nw3 = jnp.repeat(neighbor_weights.T, c, axis=1)

    bb = 8
    grid = (b // bb,)
    total = float(b * nb * c)
    body = functools.partial(_loss_body, total, nb)
    res = pl.pallas_call(
        body,
        grid=grid,
        in_specs=[
            pl.BlockSpec((nmax, n_lanes), lambda i: (0, 0)),
            pl.BlockSpec((nmax, n_lanes), lambda i: (0, 0)),
            pl.BlockSpec((bb, n_lanes), lambda i: (i, 0)),
            pl.BlockSpec((bb, n_lanes), lambda i: (i, 0)),
        ],
        out_specs=pl.BlockSpec(
            (1, 1), lambda i: (0, 0), memory_space=pltpu.SMEM
        ),
        out_shape=jax.ShapeDtypeStruct((1, 1), jnp.float32),
        scratch_shapes=[pltpu.VMEM((len(_OFFSETS), n_lanes), jnp.float32)],
    )(ni3, nw3, out2, tgt2)
    return jnp.reshape(res, ())


# speedup vs baseline: 6.8842x; 1.5295x over previous
import functools

import jax
import jax.numpy as jnp
from jax import lax
from jax.experimental import pallas as pl
from jax.experimental.pallas import tpu as pltpu
from jax.experimental.pallas import tpu_sc as plsc

_OFFSETS = (-2, -1, 1, 2)
_NSUB = 32
_LANES = 16


def _sc_band_body(nb, vs, nmax, ni_hbm, nw_hbm, w3_hbm, ni_v, nw_v, w3_v):
    wid = lax.axis_index("s") * 2 + lax.axis_index("c")
    v0 = wid * vs
    pltpu.sync_copy(ni_hbm.at[pl.ds(v0 * nmax, vs * nmax)], ni_v)
    pltpu.sync_copy(nw_hbm.at[pl.ds(v0 * nmax, vs * nmax)], nw_v)

    lanes = lax.iota(jnp.int32, _LANES)

    def chunk(c, carry):
        vloc = c * _LANES + lanes
        vabs = v0 + vloc
        accs = [jnp.zeros((_LANES,), jnp.float32) for _ in _OFFSETS]
        for k in range(nmax):
            flat = vloc * nmax + k
            u = plsc.load_gather(ni_v, [flat])
            w = plsc.load_gather(nw_v, [flat])
            diff = u - vabs
            diff = jnp.where(diff > 2, diff - nb, diff)
            diff = jnp.where(diff < -2, diff + nb, diff)
            for slot, o in enumerate(_OFFSETS):
                accs[slot] = accs[slot] + jnp.where(diff == o, w, 0.0)
        base3 = vloc * 3
        for slot in range(len(_OFFSETS)):
            for cc in range(3):
                plsc.store_scatter(
                    w3_v, [slot * 3 * vs + base3 + cc], accs[slot]
                )
        return carry

    lax.fori_loop(0, vs // _LANES, chunk, 0)
    v_pad = vs * _NSUB
    for slot in range(len(_OFFSETS)):
        pltpu.sync_copy(
            w3_v.at[pl.ds(slot * 3 * vs, 3 * vs)],
            w3_hbm.at[pl.ds(slot * 3 * v_pad + 3 * v0, 3 * vs)],
        )


def _band_weights_sc(ni, nw):
    v, nmax = ni.shape
    vs = -(-v // (_NSUB * _LANES)) * _LANES
    v_pad = vs * _NSUB
    ni_p = jnp.pad(ni, ((0, v_pad - v), (0, 0)))
    nw_p = jnp.pad(nw, ((0, v_pad - v), (0, 0)))
    mesh = plsc.VectorSubcoreMesh(core_axis_name="c", subcore_axis_name="s")
    k = functools.partial(
        pl.kernel,
        mesh=mesh,
        out_type=jax.ShapeDtypeStruct((len(_OFFSETS) * 3 * v_pad,), jnp.float32),
        scratch_types=[
            pltpu.VMEM((vs * nmax,), jnp.int32),
            pltpu.VMEM((vs * nmax,), jnp.float32),
            pltpu.VMEM((len(_OFFSETS) * 3 * vs,), jnp.float32),
        ],
        compiler_params=pltpu.CompilerParams(needs_layout_passes=False),
    )(functools.partial(_sc_band_body, v, vs, nmax))
    return k(ni_p.reshape(-1), nw_p.reshape(-1)).reshape(
        len(_OFFSETS), 3 * v_pad
    )


def _roll_lanes(x, s):
    n = x.shape[-1]
    s = s % n
    if s == 0:
        return x
    return jnp.concatenate([x[:, s:], x[:, :s]], axis=1)


def _loss_body(total, w3_ref, out_ref, tgt_ref, loss_ref):
    i = pl.program_id(0)

    @pl.when(i == 0)
    def _init():
        loss_ref[0, 0] = 0.0

    d = out_ref[...] - tgt_ref[...]
    n = d.shape[-1]
    acc = d
    for slot, o in enumerate(_OFFSETS):
        acc = acc + w3_ref[slot : slot + 1, :n] * _roll_lanes(d, 3 * o)
    loss_ref[0, 0] += jnp.sum(acc * acc)

    @pl.when(i == pl.num_programs(0) - 1)
    def _final():
        loss_ref[0, 0] = loss_ref[0, 0] / total


def kernel(out, target, neighbor_idxs, neighbor_weights):
    b, nb, c = out.shape
    n_lanes = nb * c
    out2 = out.reshape(b, n_lanes)
    tgt2 = target.reshape(b, n_lanes)
    w3 = _band_weights_sc(neighbor_idxs, neighbor_weights)

    bb = 8
    grid = (b // bb,)
    total = float(b * nb * c)
    res = pl.pallas_call(
        functools.partial(_loss_body, total),
        grid=grid,
        in_specs=[
            pl.BlockSpec(w3.shape, lambda i: (0, 0)),
            pl.BlockSpec((bb, n_lanes), lambda i: (i, 0)),
            pl.BlockSpec((bb, n_lanes), lambda i: (i, 0)),
        ],
        out_specs=pl.BlockSpec(
            (1, 1), lambda i: (0, 0), memory_space=pltpu.SMEM
        ),
        out_shape=jax.ShapeDtypeStruct((1, 1), jnp.float32),
    )(w3, out2, tgt2)
    return jnp.reshape(res, ())


# speedup vs baseline: 64.9696x; 9.4375x over previous
import functools

import jax
import jax.numpy as jnp
from jax.experimental import pallas as pl
from jax.experimental.pallas import tpu as pltpu

_OFFSETS = (-2, -1, 1, 2)


def _roll_v(x, s):
    n = x.shape[-1]
    s = s % n
    if s == 0:
        return x
    return jnp.concatenate([x[..., s:], x[..., :s]], axis=-1)


def _loss_body(total, nb, ni_ref, nw_ref, out_ref, tgt_ref, loss_ref, w_ref):
    i = pl.program_id(0)

    @pl.when(i == 0)
    def _init():
        ni = ni_ref[...]
        nw = nw_ref[...]
        vv = jax.lax.broadcasted_iota(jnp.int32, ni.shape, 1)
        for slot, o in enumerate(_OFFSETS):
            t = vv + o
            t = jnp.where(t < 0, t + nb, t)
            t = jnp.where(t >= nb, t - nb, t)
            w_ref[slot : slot + 1, :] = jnp.sum(
                jnp.where(ni == t, nw, 0.0), axis=0, keepdims=True
            )
        loss_ref[0, 0] = 0.0

    d = out_ref[...] - tgt_ref[...]
    acc = d
    for slot, o in enumerate(_OFFSETS):
        w = w_ref[slot : slot + 1, :]
        acc = acc + w[None] * _roll_v(d, o)
    loss_ref[0, 0] += jnp.sum(acc * acc)

    @pl.when(i == pl.num_programs(0) - 1)
    def _final():
        loss_ref[0, 0] = loss_ref[0, 0] / total


def kernel(out, target, neighbor_idxs, neighbor_weights):
    b, nb, c = out.shape
    nmax = neighbor_idxs.shape[1]
    out3 = jnp.transpose(out, (2, 0, 1))
    tgt3 = jnp.transpose(target, (2, 0, 1))
    ni_t = neighbor_idxs.T
    nw_t = neighbor_weights.T

    bb = 8
    grid = (b // bb,)
    total = float(b * nb * c)
    res = pl.pallas_call(
        functools.partial(_loss_body, total, nb),
        grid=grid,
        in_specs=[
            pl.BlockSpec((nmax, nb), lambda i: (0, 0)),
            pl.BlockSpec((nmax, nb), lambda i: (0, 0)),
            pl.BlockSpec((c, bb, nb), lambda i: (0, i, 0)),
            pl.BlockSpec((c, bb, nb), lambda i: (0, i, 0)),
        ],
        out_specs=pl.BlockSpec(
            (1, 1), lambda i: (0, 0), memory_space=pltpu.SMEM
        ),
        out_shape=jax.ShapeDtypeStruct((1, 1), jnp.float32),
        scratch_shapes=[pltpu.VMEM((len(_OFFSETS), nb), jnp.float32)],
    )(ni_t, nw_t, out3, tgt3)
    return jnp.reshape(res, ())
